# baseline (speedup 1.0000x reference)
import jax
import jax.numpy as jnp
from jax.experimental import pallas as pl


def _body(u_ref, i_ref, o_ref):
    o_ref[...] = jnp.sum(u_ref[...] * i_ref[...], axis=1)


def kernel(gu, gi):
    gu = jnp.squeeze(gu)
    gi = jnp.squeeze(gi)
    n, k = gu.shape
    b = 8192
    grid = pl.cdiv(n, b)
    return pl.pallas_call(
        _body,
        grid=(grid,),
        in_specs=[
            pl.BlockSpec((b, k), lambda i: (i, 0)),
            pl.BlockSpec((b, k), lambda i: (i, 0)),
        ],
        out_specs=pl.BlockSpec((b,), lambda i: (i,)),
        out_shape=jax.ShapeDtypeStruct((n,), jnp.float32),
    )(gu, gi)
